# baseline (speedup 1.0000x reference)
import functools

import jax
import jax.numpy as jnp
from jax import lax
from jax.experimental import pallas as pl
from jax.experimental.pallas import tpu as pltpu
from jax.experimental.pallas import tpu_sc as plsc

_MIN_Y = 0.2
_MAX_Y = 0.8
_MIN_MM = 0.1
_MAX_MM = 0.2
_PAD_VALUE = -80.0
_MAXY = _MAX_Y - _MAX_MM

_B, _H, _W = 64, 256, 2048
_NW = 32
_SAMPLES_PER_WORKER = _B // _NW
_K = 8


def _mask_params(b, h):
    key = jax.random.key(42)
    k1, k2, k3 = jax.random.split(key, 3)
    coin = jax.random.uniform(k1, (b,), dtype=jnp.float32)
    start_f = jax.random.uniform(k2, (b,), dtype=jnp.float32, minval=_MIN_Y, maxval=_MAXY)
    width_f = jax.random.uniform(k3, (b,), dtype=jnp.float32, minval=_MIN_MM, maxval=_MAX_MM)
    start = jnp.floor(start_f * h).astype(jnp.int32)
    width = jnp.floor(width_f * h).astype(jnp.int32)
    width = jnp.where(coin <= 1.0, width, 0)
    return start, start + width


def _sc_body(x_hbm, params_hbm, out_hbm, se_v, pad_v, sem, psem):
    wid = lax.axis_index("s") * 2 + lax.axis_index("c")

    pltpu.sync_copy(params_hbm.at[pl.ds(wid, 1)], se_v)
    pv = se_v[0, :]

    def _init(j, _):
        pad_v[0, pl.ds(j * 16, 16)] = jnp.full((16,), _PAD_VALUE, jnp.float32)
        return 0

    lax.fori_loop(0, _W // 16, _init, 0, unroll=8)

    def _drain_one():
        pltpu.make_async_copy(
            x_hbm.at[pl.ds(0, 1)], out_hbm.at[pl.ds(0, 1)], sem
        ).wait()

    for samp in range(_SAMPLES_PER_WORKER):
        i = wid * _SAMPLES_PER_WORKER + samp
        s = pv[2 * samp]
        e = pv[2 * samp + 1]
        base = i * _H

        def _row(r, _):
            fill = (r >= s) & (r < e)

            @pl.when(fill)
            def _():
                pltpu.async_copy(pad_v, out_hbm.at[pl.ds(base + r, 1)], sem)

            @pl.when(jnp.logical_not(fill))
            def _():
                pltpu.async_copy(
                    x_hbm.at[pl.ds(base + r, 1)],
                    out_hbm.at[pl.ds(base + r, 1)],
                    sem,
                )

            @pl.when((samp > 0) | (r >= _K))
            def _():
                _drain_one()

            return 0

        lax.fori_loop(0, _H, _row, 0)

    for _ in range(_K):
        _drain_one()


def kernel(x):
    b, c, h, w = x.shape
    start, end = _mask_params(b, h)
    se = jnp.stack([start, end], axis=1).reshape(_NW, 2 * _SAMPLES_PER_WORKER)
    params = jnp.zeros((_NW, 16), jnp.int32).at[:, : 2 * _SAMPLES_PER_WORKER].set(se)
    x2 = x.reshape(b * h, w)
    mesh = plsc.VectorSubcoreMesh(core_axis_name="c", subcore_axis_name="s")
    f = functools.partial(
        pl.kernel,
        out_type=jax.ShapeDtypeStruct((b * h, w), jnp.float32),
        mesh=mesh,
        scratch_types=[
            pltpu.VMEM((1, 16), jnp.int32),
            pltpu.VMEM((1, w), jnp.float32),
            pltpu.SemaphoreType.DMA,
            pltpu.SemaphoreType.DMA,
        ],
    )(_sc_body)
    out = f(x2, params)
    return out.reshape(b, c, h, w)


# speedup vs baseline: 1.0242x; 1.0242x over previous
import functools

import jax
import jax.numpy as jnp
from jax import lax
from jax.experimental import pallas as pl
from jax.experimental.pallas import tpu as pltpu
from jax.experimental.pallas import tpu_sc as plsc

_MIN_Y = 0.2
_MAX_Y = 0.8
_MIN_MM = 0.1
_MAX_MM = 0.2
_PAD_VALUE = -80.0
_MAXY = _MAX_Y - _MAX_MM

_B, _H, _W = 64, 256, 2048
_NW = 32
_SPW = _B // _NW
_BLK = 8
_NBLK = _H // _BLK
_COPY_BITS = (16, 8, 4, 2, 1)
_PAD_BITS = (4, 2, 1)
_PAD_ROWS = 16


def _mask_params(b, h):
    key = jax.random.key(42)
    k1, k2, k3 = jax.random.split(key, 3)
    coin = jax.random.uniform(k1, (b,), dtype=jnp.float32)
    start_f = jax.random.uniform(k2, (b,), dtype=jnp.float32, minval=_MIN_Y, maxval=_MAXY)
    width_f = jax.random.uniform(k3, (b,), dtype=jnp.float32, minval=_MIN_MM, maxval=_MAX_MM)
    start = jnp.floor(start_f * h).astype(jnp.int32)
    width = jnp.floor(width_f * h).astype(jnp.int32)
    width = jnp.where(coin <= 1.0, width, 0)
    return start, start + width


def _seg_dmas(nblocks, bits, issue):
    off = jnp.int32(0)
    for nb in bits:
        cond = (nblocks & nb) != 0
        issue(off, nb, cond)
        off = jnp.where(cond, off + nb, off)


def _fill_rows(buf, r0, nrows):
    def _row(r, _):
        def _col(j, _):
            buf[r, pl.ds(j * 16, 16)] = jnp.full((16,), _PAD_VALUE, jnp.float32)
            return 0

        lax.fori_loop(0, _W // 16, _col, 0, unroll=8)
        return 0

    lax.fori_loop(r0, r0 + nrows, _row, 0)


def _sc_body(x_hbm, params_hbm, out_hbm, se_v, pad_v, bb_v, csem, psem, bsem):
    wid = lax.axis_index("s") * 2 + lax.axis_index("c")

    pltpu.sync_copy(params_hbm.at[pl.ds(wid, 1)], se_v)
    pv = se_v[0, :]

    _fill_rows(pad_v, 0, _PAD_ROWS)

    def _sample(samp):
        i = wid * _SPW + samp
        s = pv[2 * samp]
        e = pv[2 * samp + 1]
        return s, e, i * _H

    def _aligned_segments(samp):
        s, e, base = _sample(samp)
        hb = s >> 3
        p0 = (s + 7) >> 3
        p1 = e >> 3
        pb = jnp.maximum(p1 - p0, 0)
        t0 = (e + 7) >> 3
        tb = _NBLK - t0
        return s, e, base, hb, p0, pb, t0, tb

    def _issue_copy(off, nb, cond, org=None):
        @pl.when(cond)
        def _():
            pltpu.async_copy(
                x_hbm.at[pl.ds(org + off * _BLK, nb * _BLK)],
                out_hbm.at[pl.ds(org + off * _BLK, nb * _BLK)],
                csem,
            )

    def _wait_copy(off, nb, cond, org=None):
        @pl.when(cond)
        def _():
            pltpu.make_async_copy(
                x_hbm.at[pl.ds(0, nb * _BLK)],
                out_hbm.at[pl.ds(0, nb * _BLK)],
                csem,
            ).wait()

    def _issue_pad(off, nb, cond, org=None):
        @pl.when(cond)
        def _():
            rows = nb * _BLK
            for k in range(0, rows, _PAD_ROWS):
                n = min(_PAD_ROWS, rows - k)
                pltpu.async_copy(
                    pad_v.at[pl.ds(0, n)],
                    out_hbm.at[pl.ds(org + off * _BLK + k, n)],
                    psem,
                )

    def _wait_pad(off, nb, cond, org=None):
        @pl.when(cond)
        def _():
            rows = nb * _BLK
            for k in range(0, rows, _PAD_ROWS):
                n = min(_PAD_ROWS, rows - k)
                pltpu.make_async_copy(
                    pad_v.at[pl.ds(0, n)],
                    out_hbm.at[pl.ds(0, n)],
                    psem,
                ).wait()

    for samp in range(_SPW):
        s, e, base, hb, p0, pb, t0, tb = _aligned_segments(samp)
        _seg_dmas(hb, _COPY_BITS, functools.partial(_issue_copy, org=base))
        _seg_dmas(tb, _COPY_BITS, functools.partial(_issue_copy, org=base + t0 * _BLK))
        _seg_dmas(pb, _PAD_BITS, functools.partial(_issue_pad, org=base + p0 * _BLK))

    bconds = []
    for samp in range(_SPW):
        s, e, base, hb, p0, pb, t0, tb = _aligned_segments(samp)
        for which, (bb, cond) in enumerate((
            (s >> 3, (s & 7) != 0),
            (e >> 3, (e & 7) != 0),
        )):
            slot = samp * 2 + which
            bconds.append(cond)

            @pl.when(cond)
            def _(bb=bb, s=s, e=e, base=base, slot=slot):
                g0 = base + bb * _BLK
                pltpu.sync_copy(x_hbm.at[pl.ds(g0, _BLK)], bb_v.at[slot])
                for r in range(_BLK):
                    gp = bb * _BLK + r
                    is_pad = (gp >= s) & (gp < e)

                    @pl.when(is_pad)
                    def _(r=r, slot=slot):
                        def _col(j, _):
                            bb_v[slot, r, pl.ds(j * 16, 16)] = jnp.full(
                                (16,), _PAD_VALUE, jnp.float32
                            )
                            return 0

                        lax.fori_loop(0, _W // 16, _col, 0, unroll=8)

                pltpu.async_copy(bb_v.at[slot], out_hbm.at[pl.ds(g0, _BLK)], bsem)

    for samp in range(_SPW):
        s, e, base, hb, p0, pb, t0, tb = _aligned_segments(samp)
        _seg_dmas(hb, _COPY_BITS, functools.partial(_wait_copy, org=base))
        _seg_dmas(tb, _COPY_BITS, functools.partial(_wait_copy, org=base + t0 * _BLK))
        _seg_dmas(pb, _PAD_BITS, functools.partial(_wait_pad, org=base + p0 * _BLK))

    for slot, cond in enumerate(bconds):
        @pl.when(cond)
        def _(slot=slot):
            pltpu.make_async_copy(
                bb_v.at[slot], out_hbm.at[pl.ds(0, _BLK)], bsem
            ).wait()


def kernel(x):
    b, c, h, w = x.shape
    start, end = _mask_params(b, h)
    se = jnp.stack([start, end], axis=1).reshape(_NW, 2 * _SPW)
    params = jnp.zeros((_NW, 16), jnp.int32).at[:, : 2 * _SPW].set(se)
    x2 = x.reshape(b * h, w)
    mesh = plsc.VectorSubcoreMesh(core_axis_name="c", subcore_axis_name="s")
    f = functools.partial(
        pl.kernel,
        out_type=jax.ShapeDtypeStruct((b * h, w), jnp.float32),
        mesh=mesh,
        scratch_types=[
            pltpu.VMEM((1, 16), jnp.int32),
            pltpu.VMEM((_PAD_ROWS, _W), jnp.float32),
            pltpu.VMEM((2 * _SPW, _BLK, _W), jnp.float32),
            pltpu.SemaphoreType.DMA,
            pltpu.SemaphoreType.DMA,
            pltpu.SemaphoreType.DMA,
        ],
    )(_sc_body)
    out = f(x2, params)
    return out.reshape(b, c, h, w)


# speedup vs baseline: 30.2366x; 29.5227x over previous
import jax
import jax.numpy as jnp
from jax import lax
from jax.experimental import pallas as pl
from jax.experimental.pallas import tpu as pltpu
from jax.experimental.pallas import tpu_sc as plsc

_MIN_Y = 0.2
_MAX_Y = 0.8
_MIN_MM = 0.1
_MAX_MM = 0.2
_PAD_VALUE = -80.0
_MAXY = _MAX_Y - _MAX_MM

_B, _H, _W = 64, 256, 2048
_NW = 32
_SPW = _B // _NW
_RPW = _SPW * _H
_CH = 8
_NCH = _RPW // _CH
_CPS = _H // _CH
_NBUF = 4
_LOOK = 2


def _mask_params(b, h):
    key = jax.random.key(42)
    k1, k2, k3 = jax.random.split(key, 3)
    coin = jax.random.uniform(k1, (b,), dtype=jnp.float32)
    start_f = jax.random.uniform(k2, (b,), dtype=jnp.float32, minval=_MIN_Y, maxval=_MAXY)
    width_f = jax.random.uniform(k3, (b,), dtype=jnp.float32, minval=_MIN_MM, maxval=_MAX_MM)
    start = jnp.floor(start_f * h).astype(jnp.int32)
    width = jnp.floor(width_f * h).astype(jnp.int32)
    width = jnp.where(coin <= 1.0, width, 0)
    return start, start + width


def _sc_body(x_hbm, params_hbm, out_hbm, se_v, buf_v, isem, osem):
    wid = lax.axis_index("s") * 2 + lax.axis_index("c")
    base = wid * _RPW

    pltpu.sync_copy(params_hbm.at[pl.ds(wid, 1)], se_v)
    pv = se_v[0, :]
    s0, e0, s1, e1 = pv[0], pv[1], pv[2], pv[3]

    def _chunk_info(c):
        r0 = (c % _CPS) * _CH
        in_second = c >= _CPS
        s = jnp.where(in_second, s1, s0)
        e = jnp.where(in_second, e1, e0)
        lo = jnp.clip(s - r0, 0, _CH)
        hi = jnp.clip(e - r0, 0, _CH)
        need_read = jnp.logical_not((lo == 0) & (hi == _CH))
        return base + c * _CH, lo, hi, need_read

    def _issue_in(c, slot):
        g0, _, _, need_read = _chunk_info(c)

        @pl.when(need_read)
        def _():
            pltpu.async_copy(x_hbm.at[pl.ds(g0, _CH)], buf_v.at[slot], isem)

    def _wait_in(c, slot):
        _, _, _, need_read = _chunk_info(c)

        @pl.when(need_read)
        def _():
            pltpu.make_async_copy(
                x_hbm.at[pl.ds(0, _CH)], buf_v.at[slot], isem
            ).wait()

    def _wait_out(slot):
        pltpu.make_async_copy(
            buf_v.at[slot], out_hbm.at[pl.ds(0, _CH)], osem
        ).wait()

    for c in range(_LOOK):
        _issue_in(c, c % _NBUF)

    def _step(c, k):
        nxt = c + _LOOK
        slot_n = (k + _LOOK) % _NBUF

        @pl.when(nxt < _NCH)
        def _():
            @pl.when(nxt >= _NBUF)
            def _():
                _wait_out(slot_n)

            _issue_in(nxt, slot_n)

        _wait_in(c, k)

        g0, lo, hi, _ = _chunk_info(c)

        def _fill_row(r, _):
            def _col(j, _):
                buf_v[k, r, pl.ds(j * 16, 16)] = jnp.full(
                    (16,), _PAD_VALUE, jnp.float32
                )
                return 0

            lax.fori_loop(0, _W // 16, _col, 0, unroll=8)
            return 0

        lax.fori_loop(lo, hi, _fill_row, 0)

        pltpu.async_copy(buf_v.at[k], out_hbm.at[pl.ds(g0, _CH)], osem)

    def _group(g, _):
        for k in range(_NBUF):
            _step(g * _NBUF + k, k)
        return 0

    lax.fori_loop(0, _NCH // _NBUF, _group, 0)

    for c in range(_NCH - _NBUF, _NCH):
        _wait_out(c % _NBUF)


def kernel(x):
    b, c, h, w = x.shape
    start, end = _mask_params(b, h)
    se = jnp.stack([start, end], axis=1).reshape(_NW, 2 * _SPW)
    params = jnp.zeros((_NW, 16), jnp.int32).at[:, : 2 * _SPW].set(se)
    x2 = x.reshape(b * h, w)
    mesh = plsc.VectorSubcoreMesh(core_axis_name="c", subcore_axis_name="s")
    f = pl.kernel(
        _sc_body,
        out_type=jax.ShapeDtypeStruct((b * h, w), jnp.float32),
        mesh=mesh,
        scratch_types=[
            pltpu.VMEM((1, 16), jnp.int32),
            pltpu.VMEM((_NBUF, _CH, _W), jnp.float32),
            pltpu.SemaphoreType.DMA,
            pltpu.SemaphoreType.DMA,
        ],
    )
    out = f(x2, params)
    return out.reshape(b, c, h, w)
